# Initial kernel scaffold; baseline (speedup 1.0000x reference)
import jax
import jax.numpy as jnp
from jax.experimental import pallas as pl


def kernel(node_features, edge_features, from_idx, to_idx, graph_idx, W_enc_n, b_enc_n, W_enc_e, b_enc_e, W_m1, b_m1, W_m2, b_m2, W_upd, b_upd, W_gate, b_gate, W_trans, b_trans, W_out, b_out):
    raise NotImplementedError("write your pallas kernel here")


# speedup vs baseline: 1.3561x; 1.3561x over previous
import functools

import jax
import jax.numpy as jnp
from jax import lax
from jax.experimental import pallas as pl
from jax.experimental.pallas import tpu as pltpu
from jax.experimental.pallas import tpu_sc as plsc

N = 10000
E = 160000
EP = 163840
D = 128
H = 256
NPAIR = 50
NACC = 10240
PAD_DST = 10200

_TILES = 32
_EPT = EP // _TILES
_CH = 128
_NCHUNK = _EPT // _CH

_MESH = plsc.VectorSubcoreMesh(core_axis_name="c", subcore_axis_name="s",
                               num_cores=2, num_subcores=16)

_F32 = jnp.float32


@functools.partial(
    pl.kernel,
    out_type=(jax.ShapeDtypeStruct((EP, H), _F32),
              jax.ShapeDtypeStruct((EP, H), _F32)),
    mesh=_MESH,
    scratch_types=[
        pltpu.VMEM((_CH,), jnp.int32),
        pltpu.VMEM((_CH,), jnp.int32),
        pltpu.VMEM((_CH, H), _F32),
        pltpu.VMEM((_CH, H), _F32),
        pltpu.SemaphoreType.DMA,
        pltpu.SemaphoreType.DMA,
    ],
)
def _sc_gather(p_hbm, q_hbm, fi_hbm, ti_hbm, g1_hbm, g2_hbm,
               i1_v, i2_v, r1_v, r2_v, s1, s2):
    wid = lax.axis_index("s") * 2 + lax.axis_index("c")
    base = wid * _EPT

    def chunk(k, carry):
        off = base + k * _CH
        pltpu.sync_copy(fi_hbm.at[pl.ds(off, _CH)], i1_v)
        pltpu.sync_copy(ti_hbm.at[pl.ds(off, _CH)], i2_v)
        c1 = pltpu.async_copy(p_hbm.at[i1_v], r1_v, s1)
        c2 = pltpu.async_copy(q_hbm.at[i2_v], r2_v, s2)
        c1.wait()
        c2.wait()
        pltpu.sync_copy(r1_v, g1_hbm.at[pl.ds(off, _CH)])
        pltpu.sync_copy(r2_v, g2_hbm.at[pl.ds(off, _CH)])
        return carry

    lax.fori_loop(0, _NCHUNK, chunk, 0)


@functools.partial(
    pl.kernel,
    out_type=jax.ShapeDtypeStruct((2, N, D), _F32),
    mesh=_MESH,
    scratch_types=[
        pltpu.VMEM((_CH,), jnp.int32),
        pltpu.VMEM((_CH, D), _F32),
        pltpu.VMEM((16, D), _F32),
        pltpu.VMEM((80, D), _F32),
        pltpu.VMEM_SHARED((NACC, D), _F32),
    ],
)
def _sc_scatter(m_hbm, ti_hbm, z_hbm, out_hbm, idx_v, rows_v, z_v, cp_v, acc_sh):
    cid = lax.axis_index("c")
    sid = lax.axis_index("s")
    wid = sid * 2 + cid

    pltpu.sync_copy(z_hbm, z_v)

    def zchunk(k, carry):
        pltpu.sync_copy(z_v, acc_sh.at[pl.ds(sid * 640 + k * 16, 16)])
        return carry

    lax.fori_loop(0, 40, zchunk, 0)
    plsc.subcore_barrier()

    def chunk(k, carry):
        off = wid * _EPT + k * _CH
        pltpu.sync_copy(ti_hbm.at[pl.ds(off, _CH)], idx_v)
        pltpu.sync_copy(m_hbm.at[pl.ds(off, _CH)], rows_v)
        pltpu.sync_copy(rows_v, acc_sh.at[idx_v], add=True)
        return carry

    lax.fori_loop(0, _NCHUNK, chunk, 0)
    plsc.subcore_barrier()

    def ochunk(k, carry):
        r0 = sid * 640 + k * 80

        @pl.when(r0 < N)
        def _():
            pltpu.sync_copy(acc_sh.at[pl.ds(r0, 80)], cp_v)
            pltpu.sync_copy(cp_v, out_hbm.at[cid, pl.ds(r0, 80)])

        return carry

    lax.fori_loop(0, 8, ochunk, 0)


_BN = 2000
_BE = 2048


def _enc_body(x, wn, bn, w1a, w1b, h, p, q):
    hv = jnp.maximum(x[...] @ wn[...] + bn[...], 0.0)
    h[...] = hv
    p[...] = hv @ w1a[...]
    q[...] = hv @ w1b[...]


_enc = pl.pallas_call(
    _enc_body,
    grid=(N // _BN,),
    in_specs=[pl.BlockSpec((_BN, D), lambda i: (i, 0)),
              pl.BlockSpec((D, D), lambda i: (0, 0)),
              pl.BlockSpec((1, D), lambda i: (0, 0)),
              pl.BlockSpec((D, H), lambda i: (0, 0)),
              pl.BlockSpec((D, H), lambda i: (0, 0))],
    out_specs=(pl.BlockSpec((_BN, D), lambda i: (i, 0)),
               pl.BlockSpec((_BN, H), lambda i: (i, 0)),
               pl.BlockSpec((_BN, H), lambda i: (i, 0))),
    out_shape=(jax.ShapeDtypeStruct((N, D), _F32),
               jax.ShapeDtypeStruct((N, H), _F32),
               jax.ShapeDtypeStruct((N, H), _F32)),
)


def _edgec_body(e, we, be, w1c, b1, c):
    enc = jnp.maximum(e[...] @ we[...] + be[...], 0.0)
    c[...] = enc @ w1c[...] + b1[...]


_edgec = pl.pallas_call(
    _edgec_body,
    grid=(EP // _BE,),
    in_specs=[pl.BlockSpec((_BE, 16), lambda i: (i, 0)),
              pl.BlockSpec((16, D), lambda i: (0, 0)),
              pl.BlockSpec((1, D), lambda i: (0, 0)),
              pl.BlockSpec((D, H), lambda i: (0, 0)),
              pl.BlockSpec((1, H), lambda i: (0, 0))],
    out_specs=pl.BlockSpec((_BE, H), lambda i: (i, 0)),
    out_shape=jax.ShapeDtypeStruct((EP, H), _F32),
)


def _msg_body(g1, g2, c, w2, b2, m):
    z = jnp.maximum(g1[...] + g2[...] + c[...], 0.0)
    m[...] = z @ w2[...] + b2[...]


_msg = pl.pallas_call(
    _msg_body,
    grid=(EP // _BE,),
    in_specs=[pl.BlockSpec((_BE, H), lambda i: (i, 0)),
              pl.BlockSpec((_BE, H), lambda i: (i, 0)),
              pl.BlockSpec((_BE, H), lambda i: (i, 0)),
              pl.BlockSpec((H, D), lambda i: (0, 0)),
              pl.BlockSpec((1, D), lambda i: (0, 0))],
    out_specs=pl.BlockSpec((_BE, D), lambda i: (i, 0)),
    out_shape=jax.ShapeDtypeStruct((EP, D), _F32),
)


def _rowsoftmax(s):
    mx = jnp.max(s, axis=1, keepdims=True)
    p = jnp.exp(s - mx)
    return p / jnp.sum(p, axis=1, keepdims=True)


def _upd_body(h, a0, a1, wh, wa, wx, bu, w1a, w1b, ho, po, qo):
    hv = h[...]
    ga = hv[:100]
    gb = hv[100:]
    dn = (((1,), (1,)), ((), ()))
    sim = lax.dot_general(ga, gb, dn, preferred_element_type=_F32)
    simt = lax.dot_general(gb, ga, dn, preferred_element_type=_F32)
    att_a = _rowsoftmax(sim) @ gb
    att_b = _rowsoftmax(simt) @ ga
    cross = jnp.concatenate([ga - att_a, gb - att_b], axis=0)
    agg = a0[...].reshape(200, D) + a1[...].reshape(200, D)
    hn = jnp.tanh(hv @ wh[...] + agg @ wa[...] + cross @ wx[...] + bu[...])
    ho[...] = hn
    po[...] = hn @ w1a[...]
    qo[...] = hn @ w1b[...]


_upd = pl.pallas_call(
    _upd_body,
    grid=(NPAIR,),
    in_specs=[pl.BlockSpec((200, D), lambda i: (i, 0)),
              pl.BlockSpec((1, 200, D), lambda i: (0, i, 0)),
              pl.BlockSpec((1, 200, D), lambda i: (1, i, 0)),
              pl.BlockSpec((D, D), lambda i: (0, 0)),
              pl.BlockSpec((D, D), lambda i: (0, 0)),
              pl.BlockSpec((D, D), lambda i: (0, 0)),
              pl.BlockSpec((1, D), lambda i: (0, 0)),
              pl.BlockSpec((D, H), lambda i: (0, 0)),
              pl.BlockSpec((D, H), lambda i: (0, 0))],
    out_specs=(pl.BlockSpec((200, D), lambda i: (i, 0)),
               pl.BlockSpec((200, H), lambda i: (i, 0)),
               pl.BlockSpec((200, H), lambda i: (i, 0))),
    out_shape=(jax.ShapeDtypeStruct((N, D), _F32),
               jax.ShapeDtypeStruct((N, H), _F32),
               jax.ShapeDtypeStruct((N, H), _F32)),
)


def _final_body(h, wg, bg, wt, bt, wo, bo, out):
    hv = h[...]
    g = jax.nn.sigmoid(hv @ wg[...] + bg[...]) * (hv @ wt[...] + bt[...])
    gva = jnp.sum(g[:100], axis=0, keepdims=True)
    gvb = jnp.sum(g[100:], axis=0, keepdims=True)
    x = gva @ wo[...] + bo[...]
    y = gvb @ wo[...] + bo[...]
    d = 2.0 * jnp.sum(jnp.maximum(x - y, 0.0)) + \
        2.0 * jnp.sum(jnp.maximum(y - x, 0.0))
    out[...] = jnp.full((1, 8, D), d, _F32)


_final = pl.pallas_call(
    _final_body,
    grid=(NPAIR,),
    in_specs=[pl.BlockSpec((200, D), lambda i: (i, 0)),
              pl.BlockSpec((D, D), lambda i: (0, 0)),
              pl.BlockSpec((1, D), lambda i: (0, 0)),
              pl.BlockSpec((D, D), lambda i: (0, 0)),
              pl.BlockSpec((1, D), lambda i: (0, 0)),
              pl.BlockSpec((D, D), lambda i: (0, 0)),
              pl.BlockSpec((1, D), lambda i: (0, 0))],
    out_specs=pl.BlockSpec((1, 8, D), lambda i: (i, 0, 0)),
    out_shape=jax.ShapeDtypeStruct((NPAIR, 8, D), _F32),
)


def kernel(node_features, edge_features, from_idx, to_idx, graph_idx,
           W_enc_n, b_enc_n, W_enc_e, b_enc_e, W_m1, b_m1, W_m2, b_m2,
           W_upd, b_upd, W_gate, b_gate, W_trans, b_trans, W_out, b_out):
    del graph_idx
    w1a, w1b, w1c = W_m1[:D], W_m1[D:2 * D], W_m1[2 * D:]
    wh, wa, wx = W_upd[:D], W_upd[D:2 * D], W_upd[2 * D:]
    bn = b_enc_n.reshape(1, D)
    be = b_enc_e.reshape(1, D)
    b1 = b_m1.reshape(1, H)
    b2 = b_m2.reshape(1, D)
    bu = b_upd.reshape(1, D)
    bg = b_gate.reshape(1, D)
    bt = b_trans.reshape(1, D)
    bo = b_out.reshape(1, D)

    npad = EP - E
    e_p = jnp.pad(edge_features, ((0, npad), (0, 0)))
    f_g = jnp.pad(from_idx.astype(jnp.int32), (0, npad))
    t_g = jnp.pad(to_idx.astype(jnp.int32), (0, npad))
    t_s = jnp.pad(to_idx.astype(jnp.int32), (0, npad), constant_values=PAD_DST)
    z16 = jnp.zeros((16, D), _F32)

    h, p, q = _enc(node_features, W_enc_n, bn, w1a, w1b)
    c = _edgec(e_p, W_enc_e, be, w1c, b1)

    for _ in range(5):
        g1, g2 = _sc_gather(p, q, f_g, t_g)
        m = _msg(g1, g2, c, W_m2, b2)
        aggp = _sc_scatter(m, t_s, z16)
        h, p, q = _upd(h, aggp, aggp, wh, wa, wx, bu, w1a, w1b)

    out = _final(h, W_gate, bg, W_trans, bt, W_out, bo)
    return out[:, 0, 0]


# speedup vs baseline: 1.5484x; 1.1418x over previous
import functools

import jax
import jax.numpy as jnp
from jax import lax
from jax.experimental import pallas as pl
from jax.experimental.pallas import tpu as pltpu
from jax.experimental.pallas import tpu_sc as plsc

N = 10000
E = 160000
EP = 163840
D = 128
H = 256
NPAIR = 50
NACC = 10240
PAD_DST = 10200

_TILES = 32
_EPT = EP // _TILES
_CH = 128
_NCHUNK = _EPT // _CH

_MESH = plsc.VectorSubcoreMesh(core_axis_name="c", subcore_axis_name="s",
                               num_cores=2, num_subcores=16)

_F32 = jnp.float32


_GCH = 64
_GNIT = _EPT // (2 * _GCH)


@functools.partial(
    pl.kernel,
    out_type=(jax.ShapeDtypeStruct((EP, H), _F32),
              jax.ShapeDtypeStruct((EP, H), _F32)),
    mesh=_MESH,
    scratch_types=[
        pltpu.VMEM((_GCH,), jnp.int32),
        pltpu.VMEM((_GCH,), jnp.int32),
        pltpu.VMEM((_GCH,), jnp.int32),
        pltpu.VMEM((_GCH,), jnp.int32),
        pltpu.VMEM((_GCH, H), _F32),
        pltpu.VMEM((_GCH, H), _F32),
        pltpu.VMEM((_GCH, H), _F32),
        pltpu.VMEM((_GCH, H), _F32),
        [pltpu.SemaphoreType.DMA] * 8,
    ],
)
def _sc_gather(p_hbm, q_hbm, fi_hbm, ti_hbm, g1_hbm, g2_hbm,
               i1a, i2a, i1b, i2b, r1a, r2a, r1b, r2b, sems):
    wid = lax.axis_index("s") * 2 + lax.axis_index("c")
    base = wid * _EPT
    sg1a, sg2a, sg1b, sg2b, sw1a, sw2a, sw1b, sw2b = sems

    def wb_a(off):
        return (pltpu.make_async_copy(r1a, g1_hbm.at[pl.ds(off, _GCH)], sw1a),
                pltpu.make_async_copy(r2a, g2_hbm.at[pl.ds(off, _GCH)], sw2a))

    def wb_b(off):
        return (pltpu.make_async_copy(r1b, g1_hbm.at[pl.ds(off, _GCH)], sw1b),
                pltpu.make_async_copy(r2b, g2_hbm.at[pl.ds(off, _GCH)], sw2b))

    def body(j, carry):
        offa = base + (2 * j) * _GCH
        offb = offa + _GCH

        @pl.when(j > 0)
        def _():
            wa1, wa2 = wb_a(offa)
            wa1.wait()
            wa2.wait()

        pltpu.sync_copy(fi_hbm.at[pl.ds(offa, _GCH)], i1a)
        pltpu.sync_copy(ti_hbm.at[pl.ds(offa, _GCH)], i2a)
        g1 = pltpu.async_copy(p_hbm.at[i1a], r1a, sg1a)
        g2 = pltpu.async_copy(q_hbm.at[i2a], r2a, sg2a)

        @pl.when(j > 0)
        def _():
            wb1, wb2 = wb_b(offb)
            wb1.wait()
            wb2.wait()

        pltpu.sync_copy(fi_hbm.at[pl.ds(offb, _GCH)], i1b)
        pltpu.sync_copy(ti_hbm.at[pl.ds(offb, _GCH)], i2b)
        g3 = pltpu.async_copy(p_hbm.at[i1b], r1b, sg1b)
        g4 = pltpu.async_copy(q_hbm.at[i2b], r2b, sg2b)

        g1.wait()
        g2.wait()
        pltpu.async_copy(r1a, g1_hbm.at[pl.ds(offa, _GCH)], sw1a)
        pltpu.async_copy(r2a, g2_hbm.at[pl.ds(offa, _GCH)], sw2a)
        g3.wait()
        g4.wait()
        pltpu.async_copy(r1b, g1_hbm.at[pl.ds(offb, _GCH)], sw1b)
        pltpu.async_copy(r2b, g2_hbm.at[pl.ds(offb, _GCH)], sw2b)
        return carry

    lax.fori_loop(0, _GNIT, body, 0)
    last = base + (2 * (_GNIT - 1)) * _GCH
    wa1, wa2 = wb_a(last)
    wb1, wb2 = wb_b(last + _GCH)
    wa1.wait()
    wa2.wait()
    wb1.wait()
    wb2.wait()


_SNIT = _EPT // (2 * _CH)


@functools.partial(
    pl.kernel,
    out_type=jax.ShapeDtypeStruct((2, N, D), _F32),
    mesh=_MESH,
    scratch_types=[
        pltpu.VMEM((_CH,), jnp.int32),
        pltpu.VMEM((_CH,), jnp.int32),
        pltpu.VMEM((_CH, D), _F32),
        pltpu.VMEM((_CH, D), _F32),
        pltpu.VMEM((80, D), _F32),
        pltpu.VMEM_SHARED((NACC, D), _F32),
        [pltpu.SemaphoreType.DMA] * 4,
    ],
)
def _sc_scatter(m_hbm, ti_hbm, z_hbm, out_hbm,
                ia, ib, ra, rb, cp_v, acc_sh, sems):
    cid = lax.axis_index("c")
    sid = lax.axis_index("s")
    wid = sid * 2 + cid
    sla, slb, ssa, ssb = sems

    pltpu.sync_copy(z_hbm, cp_v)

    def zchunk(k, carry):
        pltpu.sync_copy(cp_v, acc_sh.at[pl.ds(sid * 640 + k * 80, 80)])
        return carry

    lax.fori_loop(0, 8, zchunk, 0)
    plsc.subcore_barrier()

    base = wid * _EPT

    def body(j, carry):
        offa = base + (2 * j) * _CH
        offb = offa + _CH

        @pl.when(j > 0)
        def _():
            pltpu.make_async_copy(ra, acc_sh.at[ia], ssa).wait()

        pltpu.sync_copy(ti_hbm.at[pl.ds(offa, _CH)], ia)
        la = pltpu.async_copy(m_hbm.at[pl.ds(offa, _CH)], ra, sla)

        @pl.when(j > 0)
        def _():
            pltpu.make_async_copy(rb, acc_sh.at[ib], ssb).wait()

        pltpu.sync_copy(ti_hbm.at[pl.ds(offb, _CH)], ib)
        lb = pltpu.async_copy(m_hbm.at[pl.ds(offb, _CH)], rb, slb)

        la.wait()
        pltpu.async_copy(ra, acc_sh.at[ia], ssa, add=True)
        lb.wait()
        pltpu.async_copy(rb, acc_sh.at[ib], ssb, add=True)
        return carry

    lax.fori_loop(0, _SNIT, body, 0)
    pltpu.make_async_copy(ra, acc_sh.at[ia], ssa).wait()
    pltpu.make_async_copy(rb, acc_sh.at[ib], ssb).wait()
    plsc.subcore_barrier()

    def ochunk(k, carry):
        r0 = sid * 640 + k * 80

        @pl.when(r0 < N)
        def _():
            pltpu.sync_copy(acc_sh.at[pl.ds(r0, 80)], cp_v)
            pltpu.sync_copy(cp_v, out_hbm.at[cid, pl.ds(r0, 80)])

        return carry

    lax.fori_loop(0, 8, ochunk, 0)


_BN = 2000
_BE = 2048


def _enc_body(x, wn, bn, w1a, w1b, h, p, q):
    hv = jnp.maximum(x[...] @ wn[...] + bn[...], 0.0)
    h[...] = hv
    p[...] = hv @ w1a[...]
    q[...] = hv @ w1b[...]


_enc = pl.pallas_call(
    _enc_body,
    grid=(N // _BN,),
    in_specs=[pl.BlockSpec((_BN, D), lambda i: (i, 0)),
              pl.BlockSpec((D, D), lambda i: (0, 0)),
              pl.BlockSpec((1, D), lambda i: (0, 0)),
              pl.BlockSpec((D, H), lambda i: (0, 0)),
              pl.BlockSpec((D, H), lambda i: (0, 0))],
    out_specs=(pl.BlockSpec((_BN, D), lambda i: (i, 0)),
               pl.BlockSpec((_BN, H), lambda i: (i, 0)),
               pl.BlockSpec((_BN, H), lambda i: (i, 0))),
    out_shape=(jax.ShapeDtypeStruct((N, D), _F32),
               jax.ShapeDtypeStruct((N, H), _F32),
               jax.ShapeDtypeStruct((N, H), _F32)),
)


def _edgec_body(e, we, be, w1c, b1, c):
    enc = jnp.maximum(e[...] @ we[...] + be[...], 0.0)
    c[...] = enc @ w1c[...] + b1[...]


_edgec = pl.pallas_call(
    _edgec_body,
    grid=(EP // _BE,),
    in_specs=[pl.BlockSpec((_BE, 16), lambda i: (i, 0)),
              pl.BlockSpec((16, D), lambda i: (0, 0)),
              pl.BlockSpec((1, D), lambda i: (0, 0)),
              pl.BlockSpec((D, H), lambda i: (0, 0)),
              pl.BlockSpec((1, H), lambda i: (0, 0))],
    out_specs=pl.BlockSpec((_BE, H), lambda i: (i, 0)),
    out_shape=jax.ShapeDtypeStruct((EP, H), _F32),
)


def _msg_body(g1, g2, c, w2, b2, m):
    z = jnp.maximum(g1[...] + g2[...] + c[...], 0.0)
    m[...] = z @ w2[...] + b2[...]


_msg = pl.pallas_call(
    _msg_body,
    grid=(EP // _BE,),
    in_specs=[pl.BlockSpec((_BE, H), lambda i: (i, 0)),
              pl.BlockSpec((_BE, H), lambda i: (i, 0)),
              pl.BlockSpec((_BE, H), lambda i: (i, 0)),
              pl.BlockSpec((H, D), lambda i: (0, 0)),
              pl.BlockSpec((1, D), lambda i: (0, 0))],
    out_specs=pl.BlockSpec((_BE, D), lambda i: (i, 0)),
    out_shape=jax.ShapeDtypeStruct((EP, D), _F32),
)


def _rowsoftmax(s):
    mx = jnp.max(s, axis=1, keepdims=True)
    p = jnp.exp(s - mx)
    return p / jnp.sum(p, axis=1, keepdims=True)


def _upd_body(h, a0, a1, wh, wa, wx, bu, w1a, w1b, ho, po, qo):
    hv = h[...]
    ga = hv[:100]
    gb = hv[100:]
    dn = (((1,), (1,)), ((), ()))
    sim = lax.dot_general(ga, gb, dn, preferred_element_type=_F32)
    simt = lax.dot_general(gb, ga, dn, preferred_element_type=_F32)
    att_a = _rowsoftmax(sim) @ gb
    att_b = _rowsoftmax(simt) @ ga
    cross = jnp.concatenate([ga - att_a, gb - att_b], axis=0)
    agg = a0[...].reshape(200, D) + a1[...].reshape(200, D)
    hn = jnp.tanh(hv @ wh[...] + agg @ wa[...] + cross @ wx[...] + bu[...])
    ho[...] = hn
    po[...] = hn @ w1a[...]
    qo[...] = hn @ w1b[...]


_upd = pl.pallas_call(
    _upd_body,
    grid=(NPAIR,),
    in_specs=[pl.BlockSpec((200, D), lambda i: (i, 0)),
              pl.BlockSpec((1, 200, D), lambda i: (0, i, 0)),
              pl.BlockSpec((1, 200, D), lambda i: (1, i, 0)),
              pl.BlockSpec((D, D), lambda i: (0, 0)),
              pl.BlockSpec((D, D), lambda i: (0, 0)),
              pl.BlockSpec((D, D), lambda i: (0, 0)),
              pl.BlockSpec((1, D), lambda i: (0, 0)),
              pl.BlockSpec((D, H), lambda i: (0, 0)),
              pl.BlockSpec((D, H), lambda i: (0, 0))],
    out_specs=(pl.BlockSpec((200, D), lambda i: (i, 0)),
               pl.BlockSpec((200, H), lambda i: (i, 0)),
               pl.BlockSpec((200, H), lambda i: (i, 0))),
    out_shape=(jax.ShapeDtypeStruct((N, D), _F32),
               jax.ShapeDtypeStruct((N, H), _F32),
               jax.ShapeDtypeStruct((N, H), _F32)),
)


def _final_body(h, wg, bg, wt, bt, wo, bo, out):
    hv = h[...]
    g = jax.nn.sigmoid(hv @ wg[...] + bg[...]) * (hv @ wt[...] + bt[...])
    gva = jnp.sum(g[:100], axis=0, keepdims=True)
    gvb = jnp.sum(g[100:], axis=0, keepdims=True)
    x = gva @ wo[...] + bo[...]
    y = gvb @ wo[...] + bo[...]
    d = 2.0 * jnp.sum(jnp.maximum(x - y, 0.0)) + \
        2.0 * jnp.sum(jnp.maximum(y - x, 0.0))
    out[...] = jnp.full((1, 8, D), d, _F32)


_final = pl.pallas_call(
    _final_body,
    grid=(NPAIR,),
    in_specs=[pl.BlockSpec((200, D), lambda i: (i, 0)),
              pl.BlockSpec((D, D), lambda i: (0, 0)),
              pl.BlockSpec((1, D), lambda i: (0, 0)),
              pl.BlockSpec((D, D), lambda i: (0, 0)),
              pl.BlockSpec((1, D), lambda i: (0, 0)),
              pl.BlockSpec((D, D), lambda i: (0, 0)),
              pl.BlockSpec((1, D), lambda i: (0, 0))],
    out_specs=pl.BlockSpec((1, 8, D), lambda i: (i, 0, 0)),
    out_shape=jax.ShapeDtypeStruct((NPAIR, 8, D), _F32),
)


def kernel(node_features, edge_features, from_idx, to_idx, graph_idx,
           W_enc_n, b_enc_n, W_enc_e, b_enc_e, W_m1, b_m1, W_m2, b_m2,
           W_upd, b_upd, W_gate, b_gate, W_trans, b_trans, W_out, b_out):
    del graph_idx
    w1a, w1b, w1c = W_m1[:D], W_m1[D:2 * D], W_m1[2 * D:]
    wh, wa, wx = W_upd[:D], W_upd[D:2 * D], W_upd[2 * D:]
    bn = b_enc_n.reshape(1, D)
    be = b_enc_e.reshape(1, D)
    b1 = b_m1.reshape(1, H)
    b2 = b_m2.reshape(1, D)
    bu = b_upd.reshape(1, D)
    bg = b_gate.reshape(1, D)
    bt = b_trans.reshape(1, D)
    bo = b_out.reshape(1, D)

    npad = EP - E
    e_p = jnp.pad(edge_features, ((0, npad), (0, 0)))
    f_g = jnp.pad(from_idx.astype(jnp.int32), (0, npad))
    t_g = jnp.pad(to_idx.astype(jnp.int32), (0, npad))
    t_s = jnp.pad(to_idx.astype(jnp.int32), (0, npad), constant_values=PAD_DST)
    z80 = jnp.zeros((80, D), _F32)

    h, p, q = _enc(node_features, W_enc_n, bn, w1a, w1b)
    c = _edgec(e_p, W_enc_e, be, w1c, b1)

    for _ in range(5):
        g1, g2 = _sc_gather(p, q, f_g, t_g)
        m = _msg(g1, g2, c, W_m2, b2)
        aggp = _sc_scatter(m, t_s, z80)
        h, p, q = _upd(h, aggp, aggp, wh, wa, wx, bu, w1a, w1b)

    out = _final(h, W_gate, bg, W_trans, bt, W_out, bo)
    return out[:, 0, 0]


# speedup vs baseline: 1.8161x; 1.1729x over previous
import functools

import jax
import jax.numpy as jnp
from jax import lax
from jax.experimental import pallas as pl
from jax.experimental.pallas import tpu as pltpu
from jax.experimental.pallas import tpu_sc as plsc

N = 10000
E = 160000
EP = 163840
D = 128
H = 256
HP = 128
NPAIR = 50
NACC = 10240
PAD_DST = 10200

_TILES = 32
_EPT = EP // _TILES
_CH = 128
_NCHUNK = _EPT // _CH

_MESH = plsc.VectorSubcoreMesh(core_axis_name="c", subcore_axis_name="s",
                               num_cores=2, num_subcores=16)

_F32 = jnp.float32


_GCH = 128
_GNIT = _EPT // (2 * _GCH)


@functools.partial(
    pl.kernel,
    out_type=(jax.ShapeDtypeStruct((EP, HP), _F32),
              jax.ShapeDtypeStruct((EP, HP), _F32)),
    mesh=_MESH,
    scratch_types=[
        pltpu.VMEM((_GCH,), jnp.int32),
        pltpu.VMEM((_GCH,), jnp.int32),
        pltpu.VMEM((_GCH,), jnp.int32),
        pltpu.VMEM((_GCH,), jnp.int32),
        pltpu.VMEM((_GCH, HP), _F32),
        pltpu.VMEM((_GCH, HP), _F32),
        pltpu.VMEM((_GCH, HP), _F32),
        pltpu.VMEM((_GCH, HP), _F32),
        [pltpu.SemaphoreType.DMA] * 8,
    ],
)
def _sc_gather(p_hbm, q_hbm, fi_hbm, ti_hbm, g1_hbm, g2_hbm,
               i1a, i2a, i1b, i2b, r1a, r2a, r1b, r2b, sems):
    wid = lax.axis_index("s") * 2 + lax.axis_index("c")
    base = wid * _EPT
    sg1a, sg2a, sg1b, sg2b, sw1a, sw2a, sw1b, sw2b = sems

    def wb_a(off):
        return (pltpu.make_async_copy(r1a, g1_hbm.at[pl.ds(off, _GCH)], sw1a),
                pltpu.make_async_copy(r2a, g2_hbm.at[pl.ds(off, _GCH)], sw2a))

    def wb_b(off):
        return (pltpu.make_async_copy(r1b, g1_hbm.at[pl.ds(off, _GCH)], sw1b),
                pltpu.make_async_copy(r2b, g2_hbm.at[pl.ds(off, _GCH)], sw2b))

    def body(j, carry):
        offa = base + (2 * j) * _GCH
        offb = offa + _GCH

        @pl.when(j > 0)
        def _():
            wa1, wa2 = wb_a(offa)
            wa1.wait()
            wa2.wait()

        pltpu.sync_copy(fi_hbm.at[pl.ds(offa, _GCH)], i1a)
        pltpu.sync_copy(ti_hbm.at[pl.ds(offa, _GCH)], i2a)
        g1 = pltpu.async_copy(p_hbm.at[i1a], r1a, sg1a)
        g2 = pltpu.async_copy(q_hbm.at[i2a], r2a, sg2a)

        @pl.when(j > 0)
        def _():
            wb1, wb2 = wb_b(offb)
            wb1.wait()
            wb2.wait()

        pltpu.sync_copy(fi_hbm.at[pl.ds(offb, _GCH)], i1b)
        pltpu.sync_copy(ti_hbm.at[pl.ds(offb, _GCH)], i2b)
        g3 = pltpu.async_copy(p_hbm.at[i1b], r1b, sg1b)
        g4 = pltpu.async_copy(q_hbm.at[i2b], r2b, sg2b)

        g1.wait()
        g2.wait()
        pltpu.async_copy(r1a, g1_hbm.at[pl.ds(offa, _GCH)], sw1a)
        pltpu.async_copy(r2a, g2_hbm.at[pl.ds(offa, _GCH)], sw2a)
        g3.wait()
        g4.wait()
        pltpu.async_copy(r1b, g1_hbm.at[pl.ds(offb, _GCH)], sw1b)
        pltpu.async_copy(r2b, g2_hbm.at[pl.ds(offb, _GCH)], sw2b)
        return carry

    lax.fori_loop(0, _GNIT, body, 0)
    last = base + (2 * (_GNIT - 1)) * _GCH
    wa1, wa2 = wb_a(last)
    wb1, wb2 = wb_b(last + _GCH)
    wa1.wait()
    wa2.wait()
    wb1.wait()
    wb2.wait()


_SNIT = _EPT // (2 * _CH)


@functools.partial(
    pl.kernel,
    out_type=jax.ShapeDtypeStruct((2, N, D), _F32),
    mesh=_MESH,
    scratch_types=[
        pltpu.VMEM((_CH,), jnp.int32),
        pltpu.VMEM((_CH,), jnp.int32),
        pltpu.VMEM((_CH, D), _F32),
        pltpu.VMEM((_CH, D), _F32),
        pltpu.VMEM((80, D), _F32),
        pltpu.VMEM_SHARED((NACC, D), _F32),
        [pltpu.SemaphoreType.DMA] * 4,
    ],
)
def _sc_scatter(m_hbm, ti_hbm, z_hbm, out_hbm,
                ia, ib, ra, rb, cp_v, acc_sh, sems):
    cid = lax.axis_index("c")
    sid = lax.axis_index("s")
    wid = sid * 2 + cid
    sla, slb, ssa, ssb = sems

    pltpu.sync_copy(z_hbm, cp_v)

    def zchunk(k, carry):
        pltpu.sync_copy(cp_v, acc_sh.at[pl.ds(sid * 640 + k * 80, 80)])
        return carry

    lax.fori_loop(0, 8, zchunk, 0)
    plsc.subcore_barrier()

    base = wid * _EPT

    def body(j, carry):
        offa = base + (2 * j) * _CH
        offb = offa + _CH

        @pl.when(j > 0)
        def _():
            pltpu.make_async_copy(ra, acc_sh.at[ia], ssa).wait()

        pltpu.sync_copy(ti_hbm.at[pl.ds(offa, _CH)], ia)
        la = pltpu.async_copy(m_hbm.at[pl.ds(offa, _CH)], ra, sla)

        @pl.when(j > 0)
        def _():
            pltpu.make_async_copy(rb, acc_sh.at[ib], ssb).wait()

        pltpu.sync_copy(ti_hbm.at[pl.ds(offb, _CH)], ib)
        lb = pltpu.async_copy(m_hbm.at[pl.ds(offb, _CH)], rb, slb)

        la.wait()
        pltpu.async_copy(ra, acc_sh.at[ia], ssa, add=True)
        lb.wait()
        pltpu.async_copy(rb, acc_sh.at[ib], ssb, add=True)
        return carry

    lax.fori_loop(0, _SNIT, body, 0)
    pltpu.make_async_copy(ra, acc_sh.at[ia], ssa).wait()
    pltpu.make_async_copy(rb, acc_sh.at[ib], ssb).wait()
    plsc.subcore_barrier()

    def ochunk(k, carry):
        r0 = sid * 640 + k * 80

        @pl.when(r0 < N)
        def _():
            pltpu.sync_copy(acc_sh.at[pl.ds(r0, 80)], cp_v)
            pltpu.sync_copy(cp_v, out_hbm.at[cid, pl.ds(r0, 80)])

        return carry

    lax.fori_loop(0, 8, ochunk, 0)


_BN = 2000
_BE = 2048


def _f16bits(i):
    s = (i >> 16) & 0x8000
    ab = jnp.minimum(i & 0x7FFFFFFF, 0x477FE000)
    m = ab - 0x38000000
    r = (m + 0xFFF + ((m >> 13) & 1)) >> 13
    return jnp.where(ab < 0x38800000, 0, r) | s


def _f32val(h):
    s = (h & 0x8000) << 16
    a = h & 0x7FFF
    return lax.bitcast_convert_type(
        jnp.where(a == 0, 0, (a << 13) + 0x38000000) | s, _F32)


def _pack(x):
    i1 = lax.bitcast_convert_type(x[:, :HP], jnp.int32)
    i2 = lax.bitcast_convert_type(x[:, HP:], jnp.int32)
    return lax.bitcast_convert_type((_f16bits(i2) << 16) | _f16bits(i1), _F32)


def _unpack(w):
    iw = lax.bitcast_convert_type(w, jnp.int32)
    return _f32val(iw & 0xFFFF), _f32val((iw >> 16) & 0xFFFF)


def _enc_body(x, wn, bn, w1a, w1b, h, p, q):
    hv = jnp.maximum(x[...] @ wn[...] + bn[...], 0.0)
    h[...] = hv
    p[...] = _pack(hv @ w1a[...])
    q[...] = _pack(hv @ w1b[...])


_enc = pl.pallas_call(
    _enc_body,
    grid=(N // _BN,),
    in_specs=[pl.BlockSpec((_BN, D), lambda i: (i, 0)),
              pl.BlockSpec((D, D), lambda i: (0, 0)),
              pl.BlockSpec((1, D), lambda i: (0, 0)),
              pl.BlockSpec((D, H), lambda i: (0, 0)),
              pl.BlockSpec((D, H), lambda i: (0, 0))],
    out_specs=(pl.BlockSpec((_BN, D), lambda i: (i, 0)),
               pl.BlockSpec((_BN, HP), lambda i: (i, 0)),
               pl.BlockSpec((_BN, HP), lambda i: (i, 0))),
    out_shape=(jax.ShapeDtypeStruct((N, D), _F32),
               jax.ShapeDtypeStruct((N, HP), _F32),
               jax.ShapeDtypeStruct((N, HP), _F32)),
)


def _edgec_body(e, we, be, w1c, b1, c):
    enc = jnp.maximum(e[...] @ we[...] + be[...], 0.0)
    c[...] = enc @ w1c[...] + b1[...]


_edgec = pl.pallas_call(
    _edgec_body,
    grid=(EP // _BE,),
    in_specs=[pl.BlockSpec((_BE, 16), lambda i: (i, 0)),
              pl.BlockSpec((16, D), lambda i: (0, 0)),
              pl.BlockSpec((1, D), lambda i: (0, 0)),
              pl.BlockSpec((D, H), lambda i: (0, 0)),
              pl.BlockSpec((1, H), lambda i: (0, 0))],
    out_specs=pl.BlockSpec((_BE, H), lambda i: (i, 0)),
    out_shape=jax.ShapeDtypeStruct((EP, H), _F32),
)


def _msg_body(g1, g2, c, w2a, w2b, b2, m):
    g1a, g1b = _unpack(g1[...])
    g2a, g2b = _unpack(g2[...])
    cv = c[...]
    zlo = jnp.maximum(g1a + g2a + cv[:, :HP], 0.0)
    zhi = jnp.maximum(g1b + g2b + cv[:, HP:], 0.0)
    m[...] = zlo @ w2a[...] + zhi @ w2b[...] + b2[...]


_msg = pl.pallas_call(
    _msg_body,
    grid=(EP // _BE,),
    in_specs=[pl.BlockSpec((_BE, HP), lambda i: (i, 0)),
              pl.BlockSpec((_BE, HP), lambda i: (i, 0)),
              pl.BlockSpec((_BE, H), lambda i: (i, 0)),
              pl.BlockSpec((HP, D), lambda i: (0, 0)),
              pl.BlockSpec((HP, D), lambda i: (0, 0)),
              pl.BlockSpec((1, D), lambda i: (0, 0))],
    out_specs=pl.BlockSpec((_BE, D), lambda i: (i, 0)),
    out_shape=jax.ShapeDtypeStruct((EP, D), _F32),
)


def _rowsoftmax(s):
    mx = jnp.max(s, axis=1, keepdims=True)
    p = jnp.exp(s - mx)
    return p / jnp.sum(p, axis=1, keepdims=True)


def _upd_body(h, a0, a1, wh, wa, wx, bu, w1a, w1b, ho, po, qo):
    hv = h[...]
    ga = hv[:100]
    gb = hv[100:]
    dn = (((1,), (1,)), ((), ()))
    sim = lax.dot_general(ga, gb, dn, preferred_element_type=_F32)
    simt = lax.dot_general(gb, ga, dn, preferred_element_type=_F32)
    att_a = _rowsoftmax(sim) @ gb
    att_b = _rowsoftmax(simt) @ ga
    cross = jnp.concatenate([ga - att_a, gb - att_b], axis=0)
    agg = a0[...].reshape(200, D) + a1[...].reshape(200, D)
    hn = jnp.tanh(hv @ wh[...] + agg @ wa[...] + cross @ wx[...] + bu[...])
    ho[...] = hn
    po[...] = _pack(hn @ w1a[...])
    qo[...] = _pack(hn @ w1b[...])


_upd = pl.pallas_call(
    _upd_body,
    grid=(NPAIR,),
    in_specs=[pl.BlockSpec((200, D), lambda i: (i, 0)),
              pl.BlockSpec((1, 200, D), lambda i: (0, i, 0)),
              pl.BlockSpec((1, 200, D), lambda i: (1, i, 0)),
              pl.BlockSpec((D, D), lambda i: (0, 0)),
              pl.BlockSpec((D, D), lambda i: (0, 0)),
              pl.BlockSpec((D, D), lambda i: (0, 0)),
              pl.BlockSpec((1, D), lambda i: (0, 0)),
              pl.BlockSpec((D, H), lambda i: (0, 0)),
              pl.BlockSpec((D, H), lambda i: (0, 0))],
    out_specs=(pl.BlockSpec((200, D), lambda i: (i, 0)),
               pl.BlockSpec((200, HP), lambda i: (i, 0)),
               pl.BlockSpec((200, HP), lambda i: (i, 0))),
    out_shape=(jax.ShapeDtypeStruct((N, D), _F32),
               jax.ShapeDtypeStruct((N, HP), _F32),
               jax.ShapeDtypeStruct((N, HP), _F32)),
)


def _final_body(h, wg, bg, wt, bt, wo, bo, out):
    hv = h[...]
    g = jax.nn.sigmoid(hv @ wg[...] + bg[...]) * (hv @ wt[...] + bt[...])
    gva = jnp.sum(g[:100], axis=0, keepdims=True)
    gvb = jnp.sum(g[100:], axis=0, keepdims=True)
    x = gva @ wo[...] + bo[...]
    y = gvb @ wo[...] + bo[...]
    d = 2.0 * jnp.sum(jnp.maximum(x - y, 0.0)) + \
        2.0 * jnp.sum(jnp.maximum(y - x, 0.0))
    out[...] = jnp.full((1, 8, D), d, _F32)


_final = pl.pallas_call(
    _final_body,
    grid=(NPAIR,),
    in_specs=[pl.BlockSpec((200, D), lambda i: (i, 0)),
              pl.BlockSpec((D, D), lambda i: (0, 0)),
              pl.BlockSpec((1, D), lambda i: (0, 0)),
              pl.BlockSpec((D, D), lambda i: (0, 0)),
              pl.BlockSpec((1, D), lambda i: (0, 0)),
              pl.BlockSpec((D, D), lambda i: (0, 0)),
              pl.BlockSpec((1, D), lambda i: (0, 0))],
    out_specs=pl.BlockSpec((1, 8, D), lambda i: (i, 0, 0)),
    out_shape=jax.ShapeDtypeStruct((NPAIR, 8, D), _F32),
)


def kernel(node_features, edge_features, from_idx, to_idx, graph_idx,
           W_enc_n, b_enc_n, W_enc_e, b_enc_e, W_m1, b_m1, W_m2, b_m2,
           W_upd, b_upd, W_gate, b_gate, W_trans, b_trans, W_out, b_out):
    del graph_idx
    w1a, w1b, w1c = W_m1[:D], W_m1[D:2 * D], W_m1[2 * D:]
    wh, wa, wx = W_upd[:D], W_upd[D:2 * D], W_upd[2 * D:]
    bn = b_enc_n.reshape(1, D)
    be = b_enc_e.reshape(1, D)
    b1 = b_m1.reshape(1, H)
    b2 = b_m2.reshape(1, D)
    bu = b_upd.reshape(1, D)
    bg = b_gate.reshape(1, D)
    bt = b_trans.reshape(1, D)
    bo = b_out.reshape(1, D)

    npad = EP - E
    e_p = jnp.pad(edge_features, ((0, npad), (0, 0)))
    f_g = jnp.pad(from_idx.astype(jnp.int32), (0, npad))
    t_g = jnp.pad(to_idx.astype(jnp.int32), (0, npad))
    t_s = jnp.pad(to_idx.astype(jnp.int32), (0, npad), constant_values=PAD_DST)
    z80 = jnp.zeros((80, D), _F32)

    h, p, q = _enc(node_features, W_enc_n, bn, w1a, w1b)
    c = _edgec(e_p, W_enc_e, be, w1c, b1)

    for _ in range(5):
        g1, g2 = _sc_gather(p, q, f_g, t_g)
        m = _msg(g1, g2, c, W_m2[:HP], W_m2[HP:], b2)
        aggp = _sc_scatter(m, t_s, z80)
        h, p, q = _upd(h, aggp, aggp, wh, wa, wx, bu, w1a, w1b)

    out = _final(h, W_gate, bg, W_trans, bt, W_out, bo)
    return out[:, 0, 0]


# speedup vs baseline: 1.8492x; 1.0182x over previous
import functools

import jax
import jax.numpy as jnp
from jax import lax
from jax.experimental import pallas as pl
from jax.experimental.pallas import tpu as pltpu
from jax.experimental.pallas import tpu_sc as plsc

N = 10000
E = 160000
EP = 163840
D = 128
H = 256
HP = 128
NPAIR = 50
NACC = 10240
PAD_DST = 10200

_TILES = 32
_EPT = EP // _TILES
_CH = 128
_NCHUNK = _EPT // _CH

_MESH = plsc.VectorSubcoreMesh(core_axis_name="c", subcore_axis_name="s",
                               num_cores=2, num_subcores=16)

_F32 = jnp.float32


_GCH = 128
_GNIT = _EPT // (2 * _GCH)


@functools.partial(
    pl.kernel,
    out_type=(jax.ShapeDtypeStruct((EP, HP), _F32),
              jax.ShapeDtypeStruct((EP, HP), _F32)),
    mesh=_MESH,
    scratch_types=[
        pltpu.VMEM((_EPT,), jnp.int32),
        pltpu.VMEM((_EPT,), jnp.int32),
        pltpu.VMEM((_GCH, HP), _F32),
        pltpu.VMEM((_GCH, HP), _F32),
        pltpu.VMEM((_GCH, HP), _F32),
        pltpu.VMEM((_GCH, HP), _F32),
        [pltpu.SemaphoreType.DMA] * 8,
    ],
)
def _sc_gather(p_hbm, q_hbm, fi_hbm, ti_hbm, g1_hbm, g2_hbm,
               fi_v, ti_v, r1a, r2a, r1b, r2b, sems):
    wid = lax.axis_index("s") * 2 + lax.axis_index("c")
    base = wid * _EPT
    sg1a, sg2a, sg1b, sg2b, sw1a, sw2a, sw1b, sw2b = sems
    pltpu.sync_copy(fi_hbm.at[pl.ds(base, _EPT)], fi_v)
    pltpu.sync_copy(ti_hbm.at[pl.ds(base, _EPT)], ti_v)

    def wb_a(off):
        return (pltpu.make_async_copy(r1a, g1_hbm.at[pl.ds(off, _GCH)], sw1a),
                pltpu.make_async_copy(r2a, g2_hbm.at[pl.ds(off, _GCH)], sw2a))

    def wb_b(off):
        return (pltpu.make_async_copy(r1b, g1_hbm.at[pl.ds(off, _GCH)], sw1b),
                pltpu.make_async_copy(r2b, g2_hbm.at[pl.ds(off, _GCH)], sw2b))

    def body(j, carry):
        la = (2 * j) * _GCH
        lb = la + _GCH
        offa = base + la
        offb = base + lb

        @pl.when(j > 0)
        def _():
            wa1, wa2 = wb_a(offa)
            wa1.wait()
            wa2.wait()

        g1 = pltpu.async_copy(p_hbm.at[fi_v.at[pl.ds(la, _GCH)]], r1a, sg1a)
        g2 = pltpu.async_copy(q_hbm.at[ti_v.at[pl.ds(la, _GCH)]], r2a, sg2a)

        @pl.when(j > 0)
        def _():
            wb1, wb2 = wb_b(offb)
            wb1.wait()
            wb2.wait()

        g3 = pltpu.async_copy(p_hbm.at[fi_v.at[pl.ds(lb, _GCH)]], r1b, sg1b)
        g4 = pltpu.async_copy(q_hbm.at[ti_v.at[pl.ds(lb, _GCH)]], r2b, sg2b)

        g1.wait()
        g2.wait()
        pltpu.async_copy(r1a, g1_hbm.at[pl.ds(offa, _GCH)], sw1a)
        pltpu.async_copy(r2a, g2_hbm.at[pl.ds(offa, _GCH)], sw2a)
        g3.wait()
        g4.wait()
        pltpu.async_copy(r1b, g1_hbm.at[pl.ds(offb, _GCH)], sw1b)
        pltpu.async_copy(r2b, g2_hbm.at[pl.ds(offb, _GCH)], sw2b)
        return carry

    lax.fori_loop(0, _GNIT, body, 0)
    last = base + (2 * (_GNIT - 1)) * _GCH
    wa1, wa2 = wb_a(last)
    wb1, wb2 = wb_b(last + _GCH)
    wa1.wait()
    wa2.wait()
    wb1.wait()
    wb2.wait()


_SNIT = _EPT // (2 * _CH)


@functools.partial(
    pl.kernel,
    out_type=jax.ShapeDtypeStruct((2, N, D), _F32),
    mesh=_MESH,
    scratch_types=[
        pltpu.VMEM((_NCHUNK, _CH), jnp.int32),
        pltpu.VMEM((_CH, D), _F32),
        pltpu.VMEM((_CH, D), _F32),
        pltpu.VMEM((80, D), _F32),
        pltpu.VMEM_SHARED((NACC, D), _F32),
        [pltpu.SemaphoreType.DMA] * 4,
    ],
)
def _sc_scatter(m_hbm, ti_hbm, z_hbm, out_hbm,
                ix_v, ra, rb, cp_v, acc_sh, sems):
    cid = lax.axis_index("c")
    sid = lax.axis_index("s")
    wid = sid * 2 + cid
    sla, slb, ssa, ssb = sems

    pltpu.sync_copy(ti_hbm.at[wid], ix_v)

    pltpu.sync_copy(z_hbm, cp_v)

    def zchunk(k, carry):
        pltpu.sync_copy(cp_v, acc_sh.at[pl.ds(sid * 640 + k * 80, 80)])
        return carry

    lax.fori_loop(0, 8, zchunk, 0)
    plsc.subcore_barrier()

    base = wid * _EPT

    def body(j, carry):
        ka = 2 * j
        kb = ka + 1
        offa = base + ka * _CH
        offb = offa + _CH

        @pl.when(j > 0)
        def _():
            pltpu.make_async_copy(ra, acc_sh.at[ix_v.at[ka]], ssa).wait()

        la = pltpu.async_copy(m_hbm.at[pl.ds(offa, _CH)], ra, sla)

        @pl.when(j > 0)
        def _():
            pltpu.make_async_copy(rb, acc_sh.at[ix_v.at[kb]], ssb).wait()

        lb = pltpu.async_copy(m_hbm.at[pl.ds(offb, _CH)], rb, slb)

        la.wait()
        pltpu.async_copy(ra, acc_sh.at[ix_v.at[ka]], ssa, add=True)
        lb.wait()
        pltpu.async_copy(rb, acc_sh.at[ix_v.at[kb]], ssb, add=True)
        return carry

    lax.fori_loop(0, _SNIT, body, 0)
    pltpu.make_async_copy(ra, acc_sh.at[ix_v.at[0]], ssa).wait()
    pltpu.make_async_copy(rb, acc_sh.at[ix_v.at[1]], ssb).wait()
    plsc.subcore_barrier()

    def ochunk(k, carry):
        r0 = sid * 640 + k * 80

        @pl.when(r0 < N)
        def _():
            pltpu.sync_copy(acc_sh.at[pl.ds(r0, 80)], cp_v)
            pltpu.sync_copy(cp_v, out_hbm.at[cid, pl.ds(r0, 80)])

        return carry

    lax.fori_loop(0, 8, ochunk, 0)


def _mm(a, b):
    return jnp.dot(a, b)


_BN = 2000
_BE = 2048


def _f16bits(i):
    s = (i >> 16) & 0x8000
    ab = jnp.minimum(i & 0x7FFFFFFF, 0x477FE000)
    m = ab - 0x38000000
    r = (m + 0xFFF + ((m >> 13) & 1)) >> 13
    return jnp.where(ab < 0x38800000, 0, r) | s


def _f32val(h):
    s = (h & 0x8000) << 16
    a = h & 0x7FFF
    return lax.bitcast_convert_type(
        jnp.where(a == 0, 0, (a << 13) + 0x38000000) | s, _F32)


def _pack(x):
    i1 = lax.bitcast_convert_type(x[:, :HP], jnp.int32)
    i2 = lax.bitcast_convert_type(x[:, HP:], jnp.int32)
    return lax.bitcast_convert_type((_f16bits(i2) << 16) | _f16bits(i1), _F32)


def _unpack(w):
    iw = lax.bitcast_convert_type(w, jnp.int32)
    return _f32val(iw & 0xFFFF), _f32val((iw >> 16) & 0xFFFF)


def _enc_body(x, wn, bn, w1a, w1b, h, p, q):
    hv = jnp.maximum(_mm(x[...], wn[...]) + bn[...], 0.0)
    h[...] = hv
    p[...] = _pack(_mm(hv, w1a[...]))
    q[...] = _pack(_mm(hv, w1b[...]))


_enc = pl.pallas_call(
    _enc_body,
    grid=(N // _BN,),
    in_specs=[pl.BlockSpec((_BN, D), lambda i: (i, 0)),
              pl.BlockSpec((D, D), lambda i: (0, 0)),
              pl.BlockSpec((1, D), lambda i: (0, 0)),
              pl.BlockSpec((D, H), lambda i: (0, 0)),
              pl.BlockSpec((D, H), lambda i: (0, 0))],
    out_specs=(pl.BlockSpec((_BN, D), lambda i: (i, 0)),
               pl.BlockSpec((_BN, HP), lambda i: (i, 0)),
               pl.BlockSpec((_BN, HP), lambda i: (i, 0))),
    out_shape=(jax.ShapeDtypeStruct((N, D), _F32),
               jax.ShapeDtypeStruct((N, HP), _F32),
               jax.ShapeDtypeStruct((N, HP), _F32)),
)


def _edgec_body(e, we, be, w1c, b1, c):
    enc = jnp.maximum(_mm(e[...], we[...]) + be[...], 0.0)
    c[...] = _mm(enc, w1c[...]) + b1[...]


_edgec = pl.pallas_call(
    _edgec_body,
    grid=(EP // _BE,),
    in_specs=[pl.BlockSpec((_BE, 16), lambda i: (i, 0)),
              pl.BlockSpec((16, D), lambda i: (0, 0)),
              pl.BlockSpec((1, D), lambda i: (0, 0)),
              pl.BlockSpec((D, H), lambda i: (0, 0)),
              pl.BlockSpec((1, H), lambda i: (0, 0))],
    out_specs=pl.BlockSpec((_BE, H), lambda i: (i, 0)),
    out_shape=jax.ShapeDtypeStruct((EP, H), _F32),
)


def _msg_body(g1, g2, c, w2, b2, m):
    g1a, g1b = _unpack(g1[...])
    g2a, g2b = _unpack(g2[...])
    cv = c[...]
    zlo = jnp.maximum(g1a + g2a + cv[:, :HP], 0.0)
    zhi = jnp.maximum(g1b + g2b + cv[:, HP:], 0.0)
    z = jnp.concatenate([zlo, zhi], axis=1)
    m[...] = _mm(z, w2[...]) + b2[...]


_msg = pl.pallas_call(
    _msg_body,
    grid=(EP // _BE,),
    in_specs=[pl.BlockSpec((_BE, HP), lambda i: (i, 0)),
              pl.BlockSpec((_BE, HP), lambda i: (i, 0)),
              pl.BlockSpec((_BE, H), lambda i: (i, 0)),
              pl.BlockSpec((H, D), lambda i: (0, 0)),
              pl.BlockSpec((1, D), lambda i: (0, 0))],
    out_specs=pl.BlockSpec((_BE, D), lambda i: (i, 0)),
    out_shape=jax.ShapeDtypeStruct((EP, D), _F32),
)


def _rowsoftmax(s):
    mx = jnp.max(s, axis=1, keepdims=True)
    p = jnp.exp(s - mx)
    return p / jnp.sum(p, axis=1, keepdims=True)


def _upd_body(h, a0, a1, wh, wa, wx, bu, w1a, w1b, ho, po, qo):
    hv = h[...]
    ga = hv[:100]
    gb = hv[100:]
    dn = (((1,), (1,)), ((), ()))
    sim = lax.dot_general(ga, gb, dn, preferred_element_type=_F32)
    simt = lax.dot_general(gb, ga, dn, preferred_element_type=_F32)
    att_a = _mm(_rowsoftmax(sim), gb)
    att_b = _mm(_rowsoftmax(simt), ga)
    cross = jnp.concatenate([ga - att_a, gb - att_b], axis=0)
    agg = a0[...].reshape(200, D) + a1[...].reshape(200, D)
    upd_in = jnp.concatenate([hv, agg, cross], axis=1)
    wu = jnp.concatenate([wh[...], wa[...], wx[...]], axis=0)
    hn = jnp.tanh(_mm(upd_in, wu) + bu[...])
    ho[...] = hn
    po[...] = _pack(_mm(hn, w1a[...]))
    qo[...] = _pack(_mm(hn, w1b[...]))


_upd = pl.pallas_call(
    _upd_body,
    grid=(NPAIR,),
    in_specs=[pl.BlockSpec((200, D), lambda i: (i, 0)),
              pl.BlockSpec((1, 200, D), lambda i: (0, i, 0)),
              pl.BlockSpec((1, 200, D), lambda i: (1, i, 0)),
              pl.BlockSpec((D, D), lambda i: (0, 0)),
              pl.BlockSpec((D, D), lambda i: (0, 0)),
              pl.BlockSpec((D, D), lambda i: (0, 0)),
              pl.BlockSpec((1, D), lambda i: (0, 0)),
              pl.BlockSpec((D, H), lambda i: (0, 0)),
              pl.BlockSpec((D, H), lambda i: (0, 0))],
    out_specs=(pl.BlockSpec((200, D), lambda i: (i, 0)),
               pl.BlockSpec((200, HP), lambda i: (i, 0)),
               pl.BlockSpec((200, HP), lambda i: (i, 0))),
    out_shape=(jax.ShapeDtypeStruct((N, D), _F32),
               jax.ShapeDtypeStruct((N, HP), _F32),
               jax.ShapeDtypeStruct((N, HP), _F32)),
)


def _final_body(h, wg, bg, wt, bt, wo, bo, out):
    hv = h[...]
    g = jax.nn.sigmoid(_mm(hv, wg[...]) + bg[...]) * (_mm(hv, wt[...]) + bt[...])
    gva = jnp.sum(g[:100], axis=0, keepdims=True)
    gvb = jnp.sum(g[100:], axis=0, keepdims=True)
    x = _mm(gva, wo[...]) + bo[...]
    y = _mm(gvb, wo[...]) + bo[...]
    d = 2.0 * jnp.sum(jnp.maximum(x - y, 0.0)) + \
        2.0 * jnp.sum(jnp.maximum(y - x, 0.0))
    out[...] = jnp.full((1, 8, D), d, _F32)


_final = pl.pallas_call(
    _final_body,
    grid=(NPAIR,),
    in_specs=[pl.BlockSpec((200, D), lambda i: (i, 0)),
              pl.BlockSpec((D, D), lambda i: (0, 0)),
              pl.BlockSpec((1, D), lambda i: (0, 0)),
              pl.BlockSpec((D, D), lambda i: (0, 0)),
              pl.BlockSpec((1, D), lambda i: (0, 0)),
              pl.BlockSpec((D, D), lambda i: (0, 0)),
              pl.BlockSpec((1, D), lambda i: (0, 0))],
    out_specs=pl.BlockSpec((1, 8, D), lambda i: (i, 0, 0)),
    out_shape=jax.ShapeDtypeStruct((NPAIR, 8, D), _F32),
)


def kernel(node_features, edge_features, from_idx, to_idx, graph_idx,
           W_enc_n, b_enc_n, W_enc_e, b_enc_e, W_m1, b_m1, W_m2, b_m2,
           W_upd, b_upd, W_gate, b_gate, W_trans, b_trans, W_out, b_out):
    del graph_idx
    w1a, w1b, w1c = W_m1[:D], W_m1[D:2 * D], W_m1[2 * D:]
    wh, wa, wx = W_upd[:D], W_upd[D:2 * D], W_upd[2 * D:]
    bn = b_enc_n.reshape(1, D)
    be = b_enc_e.reshape(1, D)
    b1 = b_m1.reshape(1, H)
    b2 = b_m2.reshape(1, D)
    bu = b_upd.reshape(1, D)
    bg = b_gate.reshape(1, D)
    bt = b_trans.reshape(1, D)
    bo = b_out.reshape(1, D)

    npad = EP - E
    e_p = jnp.pad(edge_features, ((0, npad), (0, 0)))
    f_g = jnp.pad(from_idx.astype(jnp.int32), (0, npad))
    t_g = jnp.pad(to_idx.astype(jnp.int32), (0, npad))
    t_s = jnp.pad(to_idx.astype(jnp.int32), (0, npad),
                  constant_values=PAD_DST).reshape(_TILES, _NCHUNK, _CH)
    z80 = jnp.zeros((80, D), _F32)

    h, p, q = _enc(node_features, W_enc_n, bn, w1a, w1b)
    c = _edgec(e_p, W_enc_e, be, w1c, b1)

    for _ in range(5):
        g1, g2 = _sc_gather(p, q, f_g, t_g)
        m = _msg(g1, g2, c, W_m2, b2)
        aggp = _sc_scatter(m, t_s, z80)
        h, p, q = _upd(h, aggp, aggp, wh, wa, wx, bu, w1a, w1b)

    out = _final(h, W_gate, bg, W_trans, bt, W_out, bo)
    return out[:, 0, 0]


# speedup vs baseline: 1.8617x; 1.0067x over previous
import functools

import jax
import jax.numpy as jnp
from jax import lax
from jax.experimental import pallas as pl
from jax.experimental.pallas import tpu as pltpu
from jax.experimental.pallas import tpu_sc as plsc

N = 10000
E = 160000
EP = 163840
D = 128
H = 256
HP = 128
NPAIR = 50
NACC = 10240
PAD_DST = 10200

_TILES = 32
_EPT = EP // _TILES
_CH = 128
_NCHUNK = _EPT // _CH

_MESH = plsc.VectorSubcoreMesh(core_axis_name="c", subcore_axis_name="s",
                               num_cores=2, num_subcores=16)

_F32 = jnp.float32


_GCH = 128
NSLICE = 2
EP2 = EP // NSLICE
_EPT2 = EP2 // _TILES
_GNIT2 = _EPT2 // (2 * _GCH)


def _make_gather(sl):
    gbase0 = sl * EP2

    @functools.partial(
        pl.kernel,
        out_type=(jax.ShapeDtypeStruct((EP2, HP), _F32),
                  jax.ShapeDtypeStruct((EP2, HP), _F32)),
        mesh=_MESH,
        scratch_types=[
            pltpu.VMEM((_EPT2,), jnp.int32),
            pltpu.VMEM((_EPT2,), jnp.int32),
            pltpu.VMEM((_GCH, HP), _F32),
            pltpu.VMEM((_GCH, HP), _F32),
            pltpu.VMEM((_GCH, HP), _F32),
            pltpu.VMEM((_GCH, HP), _F32),
            [pltpu.SemaphoreType.DMA] * 8,
        ],
    )
    def gather(p_hbm, q_hbm, fi_hbm, ti_hbm, dep_hbm, g1_hbm, g2_hbm,
               fi_v, ti_v, r1a, r2a, r1b, r2b, sems):
        del dep_hbm
        wid = lax.axis_index("s") * 2 + lax.axis_index("c")
        base = wid * _EPT2
        sg1a, sg2a, sg1b, sg2b, sw1a, sw2a, sw1b, sw2b = sems
        pltpu.sync_copy(fi_hbm.at[pl.ds(gbase0 + base, _EPT2)], fi_v)
        pltpu.sync_copy(ti_hbm.at[pl.ds(gbase0 + base, _EPT2)], ti_v)

        def wb_a(off):
            return (pltpu.make_async_copy(r1a, g1_hbm.at[pl.ds(off, _GCH)], sw1a),
                    pltpu.make_async_copy(r2a, g2_hbm.at[pl.ds(off, _GCH)], sw2a))

        def wb_b(off):
            return (pltpu.make_async_copy(r1b, g1_hbm.at[pl.ds(off, _GCH)], sw1b),
                    pltpu.make_async_copy(r2b, g2_hbm.at[pl.ds(off, _GCH)], sw2b))

        def body(j, carry):
            la = (2 * j) * _GCH
            lb = la + _GCH
            offa = base + la
            offb = base + lb

            @pl.when(j > 0)
            def _():
                wa1, wa2 = wb_a(offa)
                wa1.wait()
                wa2.wait()

            g1 = pltpu.async_copy(p_hbm.at[fi_v.at[pl.ds(la, _GCH)]], r1a, sg1a)
            g2 = pltpu.async_copy(q_hbm.at[ti_v.at[pl.ds(la, _GCH)]], r2a, sg2a)

            @pl.when(j > 0)
            def _():
                wb1, wb2 = wb_b(offb)
                wb1.wait()
                wb2.wait()

            g3 = pltpu.async_copy(p_hbm.at[fi_v.at[pl.ds(lb, _GCH)]], r1b, sg1b)
            g4 = pltpu.async_copy(q_hbm.at[ti_v.at[pl.ds(lb, _GCH)]], r2b, sg2b)

            g1.wait()
            g2.wait()
            pltpu.async_copy(r1a, g1_hbm.at[pl.ds(offa, _GCH)], sw1a)
            pltpu.async_copy(r2a, g2_hbm.at[pl.ds(offa, _GCH)], sw2a)
            g3.wait()
            g4.wait()
            pltpu.async_copy(r1b, g1_hbm.at[pl.ds(offb, _GCH)], sw1b)
            pltpu.async_copy(r2b, g2_hbm.at[pl.ds(offb, _GCH)], sw2b)
            return carry

        lax.fori_loop(0, _GNIT2, body, 0)
        last = base + (2 * (_GNIT2 - 1)) * _GCH
        wa1, wa2 = wb_a(last)
        wb1, wb2 = wb_b(last + _GCH)
        wa1.wait()
        wa2.wait()
        wb1.wait()
        wb2.wait()

    return gather


_sc_gather0 = _make_gather(0)
_sc_gather1 = _make_gather(1)


_SNIT2 = _EPT2 // (2 * _CH)
_NCHUNK2 = _EPT2 // _CH


def _make_scatter(first):
    @functools.partial(
        pl.kernel,
        out_type=jax.ShapeDtypeStruct((2, N, D), _F32),
        mesh=_MESH,
        scratch_types=[
            pltpu.VMEM((_NCHUNK2, _CH), jnp.int32),
            pltpu.VMEM((_CH, D), _F32),
            pltpu.VMEM((_CH, D), _F32),
            pltpu.VMEM((80, D), _F32),
            pltpu.VMEM_SHARED((NACC, D), _F32),
            [pltpu.SemaphoreType.DMA] * 4,
        ],
    )
    def scatter(m_hbm, ti_hbm, z_hbm, dep_hbm, out_hbm,
                ix_v, ra, rb, cp_v, acc_sh, sems):
        del dep_hbm
        cid = lax.axis_index("c")
        sid = lax.axis_index("s")
        wid = sid * 2 + cid
        sla, slb, ssa, ssb = sems

        pltpu.sync_copy(ti_hbm.at[wid], ix_v)

        if first:
            pltpu.sync_copy(z_hbm, cp_v)

            def zchunk(k, carry):
                pltpu.sync_copy(cp_v, acc_sh.at[pl.ds(sid * 640 + k * 80, 80)])
                return carry

            lax.fori_loop(0, 8, zchunk, 0)
        else:
            def lchunk(k, carry):
                r0 = sid * 640 + k * 80

                @pl.when(r0 < N)
                def _():
                    pltpu.sync_copy(z_hbm.at[cid, pl.ds(r0, 80)], cp_v)
                    pltpu.sync_copy(cp_v, acc_sh.at[pl.ds(r0, 80)])

                return carry

            lax.fori_loop(0, 8, lchunk, 0)
        plsc.subcore_barrier()

        base = wid * _EPT2

        def body(j, carry):
            ka = 2 * j
            kb = ka + 1
            offa = base + ka * _CH
            offb = offa + _CH

            @pl.when(j > 0)
            def _():
                pltpu.make_async_copy(ra, acc_sh.at[ix_v.at[ka]], ssa).wait()

            la = pltpu.async_copy(m_hbm.at[pl.ds(offa, _CH)], ra, sla)

            @pl.when(j > 0)
            def _():
                pltpu.make_async_copy(rb, acc_sh.at[ix_v.at[kb]], ssb).wait()

            lb = pltpu.async_copy(m_hbm.at[pl.ds(offb, _CH)], rb, slb)

            la.wait()
            pltpu.async_copy(ra, acc_sh.at[ix_v.at[ka]], ssa, add=True)
            lb.wait()
            pltpu.async_copy(rb, acc_sh.at[ix_v.at[kb]], ssb, add=True)
            return carry

        lax.fori_loop(0, _SNIT2, body, 0)
        pltpu.make_async_copy(ra, acc_sh.at[ix_v.at[0]], ssa).wait()
        pltpu.make_async_copy(rb, acc_sh.at[ix_v.at[1]], ssb).wait()
        plsc.subcore_barrier()

        def ochunk(k, carry):
            r0 = sid * 640 + k * 80

            @pl.when(r0 < N)
            def _():
                pltpu.sync_copy(acc_sh.at[pl.ds(r0, 80)], cp_v)
                pltpu.sync_copy(cp_v, out_hbm.at[cid, pl.ds(r0, 80)])

            return carry

        lax.fori_loop(0, 8, ochunk, 0)

    return scatter


_sc_scatter0 = _make_scatter(True)
_sc_scatter1 = _make_scatter(False)


def _mm(a, b):
    return jnp.dot(a, b)


_BN = 2000
_BE = 2048


def _f16bits(i):
    s = (i >> 16) & 0x8000
    ab = jnp.minimum(i & 0x7FFFFFFF, 0x477FE000)
    m = ab - 0x38000000
    r = (m + 0xFFF + ((m >> 13) & 1)) >> 13
    return jnp.where(ab < 0x38800000, 0, r) | s


def _f32val(h):
    s = (h & 0x8000) << 16
    a = h & 0x7FFF
    return lax.bitcast_convert_type(
        jnp.where(a == 0, 0, (a << 13) + 0x38000000) | s, _F32)


def _pack(x):
    i1 = lax.bitcast_convert_type(x[:, :HP], jnp.int32)
    i2 = lax.bitcast_convert_type(x[:, HP:], jnp.int32)
    return lax.bitcast_convert_type((_f16bits(i2) << 16) | _f16bits(i1), _F32)


def _unpack(w):
    iw = lax.bitcast_convert_type(w, jnp.int32)
    return _f32val(iw & 0xFFFF), _f32val((iw >> 16) & 0xFFFF)


def _enc_body(x, wn, bn, w1a, w1b, h, p, q):
    hv = jnp.maximum(_mm(x[...], wn[...]) + bn[...], 0.0)
    h[...] = hv
    p[...] = _pack(_mm(hv, w1a[...]))
    q[...] = _pack(_mm(hv, w1b[...]))


_enc = pl.pallas_call(
    _enc_body,
    grid=(N // _BN,),
    in_specs=[pl.BlockSpec((_BN, D), lambda i: (i, 0)),
              pl.BlockSpec((D, D), lambda i: (0, 0)),
              pl.BlockSpec((1, D), lambda i: (0, 0)),
              pl.BlockSpec((D, H), lambda i: (0, 0)),
              pl.BlockSpec((D, H), lambda i: (0, 0))],
    out_specs=(pl.BlockSpec((_BN, D), lambda i: (i, 0)),
               pl.BlockSpec((_BN, HP), lambda i: (i, 0)),
               pl.BlockSpec((_BN, HP), lambda i: (i, 0))),
    out_shape=(jax.ShapeDtypeStruct((N, D), _F32),
               jax.ShapeDtypeStruct((N, HP), _F32),
               jax.ShapeDtypeStruct((N, HP), _F32)),
)


def _edgec_body(e, we, be, w1c, b1, c):
    enc = jnp.maximum(_mm(e[...], we[...]) + be[...], 0.0)
    c[...] = _mm(enc, w1c[...]) + b1[...]


_edgec = pl.pallas_call(
    _edgec_body,
    grid=(EP // _BE,),
    in_specs=[pl.BlockSpec((_BE, 16), lambda i: (i, 0)),
              pl.BlockSpec((16, D), lambda i: (0, 0)),
              pl.BlockSpec((1, D), lambda i: (0, 0)),
              pl.BlockSpec((D, H), lambda i: (0, 0)),
              pl.BlockSpec((1, H), lambda i: (0, 0))],
    out_specs=pl.BlockSpec((_BE, H), lambda i: (i, 0)),
    out_shape=jax.ShapeDtypeStruct((EP, H), _F32),
)


def _msg_body(g1, g2, c, w2, b2, m):
    g1a, g1b = _unpack(g1[...])
    g2a, g2b = _unpack(g2[...])
    cv = c[...]
    zlo = jnp.maximum(g1a + g2a + cv[:, :HP], 0.0)
    zhi = jnp.maximum(g1b + g2b + cv[:, HP:], 0.0)
    z = jnp.concatenate([zlo, zhi], axis=1)
    m[...] = _mm(z, w2[...]) + b2[...]


def _make_msg(sl):
    coff = sl * (EP2 // _BE)
    return pl.pallas_call(
        _msg_body,
        grid=(EP2 // _BE,),
        in_specs=[pl.BlockSpec((_BE, HP), lambda i: (i, 0)),
                  pl.BlockSpec((_BE, HP), lambda i: (i, 0)),
                  pl.BlockSpec((_BE, H), lambda i: (i + coff, 0)),
                  pl.BlockSpec((H, D), lambda i: (0, 0)),
                  pl.BlockSpec((1, D), lambda i: (0, 0))],
        out_specs=pl.BlockSpec((_BE, D), lambda i: (i, 0)),
        out_shape=jax.ShapeDtypeStruct((EP2, D), _F32),
    )


_msg0 = _make_msg(0)
_msg1 = _make_msg(1)


def _rowsoftmax(s):
    mx = jnp.max(s, axis=1, keepdims=True)
    p = jnp.exp(s - mx)
    return p / jnp.sum(p, axis=1, keepdims=True)


def _upd_body(h, a0, a1, wh, wa, wx, bu, w1a, w1b, ho, po, qo):
    hv = h[...]
    ga = hv[:100]
    gb = hv[100:]
    dn = (((1,), (1,)), ((), ()))
    sim = lax.dot_general(ga, gb, dn, preferred_element_type=_F32)
    simt = lax.dot_general(gb, ga, dn, preferred_element_type=_F32)
    att_a = _mm(_rowsoftmax(sim), gb)
    att_b = _mm(_rowsoftmax(simt), ga)
    cross = jnp.concatenate([ga - att_a, gb - att_b], axis=0)
    agg = a0[...].reshape(200, D) + a1[...].reshape(200, D)
    upd_in = jnp.concatenate([hv, agg, cross], axis=1)
    wu = jnp.concatenate([wh[...], wa[...], wx[...]], axis=0)
    hn = jnp.tanh(_mm(upd_in, wu) + bu[...])
    ho[...] = hn
    po[...] = _pack(_mm(hn, w1a[...]))
    qo[...] = _pack(_mm(hn, w1b[...]))


_upd = pl.pallas_call(
    _upd_body,
    grid=(NPAIR,),
    in_specs=[pl.BlockSpec((200, D), lambda i: (i, 0)),
              pl.BlockSpec((1, 200, D), lambda i: (0, i, 0)),
              pl.BlockSpec((1, 200, D), lambda i: (1, i, 0)),
              pl.BlockSpec((D, D), lambda i: (0, 0)),
              pl.BlockSpec((D, D), lambda i: (0, 0)),
              pl.BlockSpec((D, D), lambda i: (0, 0)),
              pl.BlockSpec((1, D), lambda i: (0, 0)),
              pl.BlockSpec((D, H), lambda i: (0, 0)),
              pl.BlockSpec((D, H), lambda i: (0, 0))],
    out_specs=(pl.BlockSpec((200, D), lambda i: (i, 0)),
               pl.BlockSpec((200, HP), lambda i: (i, 0)),
               pl.BlockSpec((200, HP), lambda i: (i, 0))),
    out_shape=(jax.ShapeDtypeStruct((N, D), _F32),
               jax.ShapeDtypeStruct((N, HP), _F32),
               jax.ShapeDtypeStruct((N, HP), _F32)),
)


def _final_body(h, wg, bg, wt, bt, wo, bo, out):
    hv = h[...]
    g = jax.nn.sigmoid(_mm(hv, wg[...]) + bg[...]) * (_mm(hv, wt[...]) + bt[...])
    gva = jnp.sum(g[:100], axis=0, keepdims=True)
    gvb = jnp.sum(g[100:], axis=0, keepdims=True)
    x = _mm(gva, wo[...]) + bo[...]
    y = _mm(gvb, wo[...]) + bo[...]
    d = 2.0 * jnp.sum(jnp.maximum(x - y, 0.0)) + \
        2.0 * jnp.sum(jnp.maximum(y - x, 0.0))
    out[...] = jnp.full((1, 8, D), d, _F32)


_final = pl.pallas_call(
    _final_body,
    grid=(NPAIR,),
    in_specs=[pl.BlockSpec((200, D), lambda i: (i, 0)),
              pl.BlockSpec((D, D), lambda i: (0, 0)),
              pl.BlockSpec((1, D), lambda i: (0, 0)),
              pl.BlockSpec((D, D), lambda i: (0, 0)),
              pl.BlockSpec((1, D), lambda i: (0, 0)),
              pl.BlockSpec((D, D), lambda i: (0, 0)),
              pl.BlockSpec((1, D), lambda i: (0, 0))],
    out_specs=pl.BlockSpec((1, 8, D), lambda i: (i, 0, 0)),
    out_shape=jax.ShapeDtypeStruct((NPAIR, 8, D), _F32),
)


def kernel(node_features, edge_features, from_idx, to_idx, graph_idx,
           W_enc_n, b_enc_n, W_enc_e, b_enc_e, W_m1, b_m1, W_m2, b_m2,
           W_upd, b_upd, W_gate, b_gate, W_trans, b_trans, W_out, b_out):
    del graph_idx
    w1a, w1b, w1c = W_m1[:D], W_m1[D:2 * D], W_m1[2 * D:]
    wh, wa, wx = W_upd[:D], W_upd[D:2 * D], W_upd[2 * D:]
    bn = b_enc_n.reshape(1, D)
    be = b_enc_e.reshape(1, D)
    b1 = b_m1.reshape(1, H)
    b2 = b_m2.reshape(1, D)
    bu = b_upd.reshape(1, D)
    bg = b_gate.reshape(1, D)
    bt = b_trans.reshape(1, D)
    bo = b_out.reshape(1, D)

    npad = EP - E
    e_p = jnp.pad(edge_features, ((0, npad), (0, 0)))
    f_g = jnp.pad(from_idx.astype(jnp.int32), (0, npad))
    t_g = jnp.pad(to_idx.astype(jnp.int32), (0, npad))
    t_s = jnp.pad(to_idx.astype(jnp.int32), (0, npad),
                  constant_values=PAD_DST)
    t_s0 = t_s[:EP2].reshape(_TILES, _NCHUNK2, _CH)
    t_s1 = t_s[EP2:].reshape(_TILES, _NCHUNK2, _CH)
    z80 = jnp.zeros((80, D), _F32)

    h, p, q = _enc(node_features, W_enc_n, bn, w1a, w1b)
    c = _edgec(e_p, W_enc_e, be, w1c, b1)

    for _ in range(5):
        g1a, g2a = _sc_gather0(p, q, f_g, t_g, z80)
        g1b, g2b = _sc_gather1(p, q, f_g, t_g, g1a)
        m0 = _msg0(g1a, g2a, c, W_m2, b2)
        m1 = _msg1(g1b, g2b, c, W_m2, b2)
        part = _sc_scatter0(m0, t_s0, z80, g1b)
        aggp = _sc_scatter1(m1, t_s1, part, part)
        h, p, q = _upd(h, aggp, aggp, wh, wa, wx, bu, w1a, w1b)

    out = _final(h, W_gate, bg, W_trans, bt, W_out, bo)
    return out[:, 0, 0]
